# initial kernel scaffold (unmeasured)
import jax
import jax.numpy as jnp
from jax import lax
from jax.experimental import pallas as pl
from jax.experimental.pallas import tpu as pltpu

N_DEV = 4
SCALE = 0.08838834764831843


def _body(x_ref, wq_ref, wo_ref, k_hbm, v_hbm, out_ref,
          wq_buf, wo_buf, k_stage, v_stage,
          wq_send, wq_recv, wo_send, wo_recv, ksem, vsem):
    my = lax.axis_index("i")
    right = lax.rem(my + 1, N_DEV)
    left = lax.rem(my + 3, N_DEV)

    wq_buf[0] = wq_ref[...]
    wo_buf[0] = wo_ref[...]
    out_ref[...] = jnp.zeros_like(out_ref)

    barrier = pltpu.get_barrier_semaphore()
    for nbr in (left, right):
        pl.semaphore_signal(barrier, inc=1, device_id=(nbr,),
                            device_id_type=pl.DeviceIdType.MESH)
    pl.semaphore_wait(barrier, 2)

    def hop(h):
        wq_rdma = pltpu.make_async_remote_copy(
            src_ref=wq_buf.at[(4 - h) % 4],
            dst_ref=wq_buf.at[(3 - h) % 4],
            send_sem=wq_send.at[h],
            recv_sem=wq_recv.at[h],
            device_id=(right,),
            device_id_type=pl.DeviceIdType.MESH,
        )
        wo_rdma = pltpu.make_async_remote_copy(
            src_ref=wo_buf.at[h],
            dst_ref=wo_buf.at[h + 1],
            send_sem=wo_send.at[h],
            recv_sem=wo_recv.at[h],
            device_id=(left,),
            device_id_type=pl.DeviceIdType.MESH,
        )
        wq_rdma.start()
        wo_rdma.start()
        return wq_rdma, wo_rdma

    def compute(s):
        g = lax.rem(my + s, N_DEV)
        h0 = g * 8
        qg = jnp.dot(x_ref[...], wq_buf[s],
                     preferred_element_type=jnp.bfloat16)
        qg5 = qg.reshape(8, 4, 64, 8, 128)
        wo3 = wo_buf[s].reshape(8, 128, 1024)
        for r in range(4):
            ck = pltpu.make_async_copy(
                k_hbm.at[my, :, r, :, pl.ds(h0, 8), :], k_stage, ksem)
            cv = pltpu.make_async_copy(
                v_hbm.at[my, :, r, :, pl.ds(h0, 8), :], v_stage, vsem)
            ck.start()
            cv.start()
            ck.wait()
            cv.wait()
            o = jnp.zeros((512, 1024), jnp.float32)
            for hb in range(2):
                hs = slice(hb * 4, hb * 4 + 4)
                qr = jnp.swapaxes(
                    qg5[:, r, :, hs, :].reshape(512, 4, 128), 0, 1)
                kr = jnp.swapaxes(
                    k_stage[:, :, hs, :].astype(jnp.bfloat16)
                    .reshape(512, 4, 128), 0, 1)
                vr = jnp.swapaxes(
                    v_stage[:, :, hs, :].astype(jnp.bfloat16)
                    .reshape(512, 4, 128), 0, 1)
                sc = lax.dot_general(
                    qr, kr, (((2,), (2,)), ((0,), (0,))),
                    preferred_element_type=jnp.float32) * SCALE
                m = jnp.max(sc, axis=-1, keepdims=True)
                e = jnp.exp(sc - m)
                den = jnp.sum(e, axis=-1, keepdims=True)
                w = (e / den).astype(jnp.bfloat16)
                ctx = lax.dot_general(
                    w, vr, (((2,), (1,)), ((0,), (0,))),
                    preferred_element_type=jnp.bfloat16)
                for hh in range(4):
                    o = o + jnp.dot(ctx[hh], wo3[hb * 4 + hh],
                                    preferred_element_type=jnp.float32)
            out_ref[:, r] = out_ref[:, r] + o.reshape(8, 64, 1024)

    h0p = hop(0)
    compute(0)
    h0p[0].wait()
    h0p[1].wait()
    h1p = hop(1)
    h1p[0].wait()
    h1p[1].wait()
    h2p = hop(2)
    compute(2)
    h2p[0].wait()
    h2p[1].wait()
    compute(1)
    compute(3)


def kernel(x, Wq, K_ext, V_ext, Wo):
    xb = x[0].astype(jnp.bfloat16)
    wqb = Wq.astype(jnp.bfloat16)
    wob = Wo.astype(jnp.bfloat16)
    k6 = K_ext.reshape(4, 8, 4, 64, 32, 128)
    v6 = V_ext.reshape(4, 8, 4, 64, 32, 128)

    out = pl.pallas_call(
        _body,
        out_shape=jax.ShapeDtypeStruct((8, 4, 64, 1024), jnp.float32),
        in_specs=[
            pl.BlockSpec(memory_space=pltpu.MemorySpace.VMEM),
            pl.BlockSpec(memory_space=pltpu.MemorySpace.VMEM),
            pl.BlockSpec(memory_space=pltpu.MemorySpace.VMEM),
            pl.BlockSpec(memory_space=pltpu.MemorySpace.ANY),
            pl.BlockSpec(memory_space=pltpu.MemorySpace.ANY),
        ],
        out_specs=pl.BlockSpec(memory_space=pltpu.MemorySpace.VMEM),
        scratch_shapes=[
            pltpu.VMEM((4, 1024, 1024), jnp.bfloat16),
            pltpu.VMEM((4, 1024, 1024), jnp.bfloat16),
            pltpu.VMEM((8, 64, 8, 128), jnp.float32),
            pltpu.VMEM((8, 64, 8, 128), jnp.float32),
            pltpu.SemaphoreType.DMA((3,)),
            pltpu.SemaphoreType.DMA((3,)),
            pltpu.SemaphoreType.DMA((3,)),
            pltpu.SemaphoreType.DMA((3,)),
            pltpu.SemaphoreType.DMA,
            pltpu.SemaphoreType.DMA,
        ],
        compiler_params=pltpu.CompilerParams(collective_id=0),
    )(xb, wqb, wob, k6, v6)
    return out.reshape(1, 2048, 1024)


# baseline (device time: 212544 ns/iter reference)
import jax
import jax.numpy as jnp
from jax import lax
from jax.experimental import pallas as pl
from jax.experimental.pallas import tpu as pltpu

N_DEV = 4
SCALE = 0.08838834764831843


def _body(x_ref, wq_ref, wo_ref, k_hbm, v_hbm, out_ref,
          wq_buf, wo_buf, k_stage, v_stage,
          wq_send, wq_recv, wo_send, wo_recv, ksem, vsem):
    my = lax.axis_index("i")
    right = lax.rem(my + 1, N_DEV)
    left = lax.rem(my + 3, N_DEV)

    wq_buf[0] = wq_ref[...]
    wo_buf[0] = wo_ref[...]
    out_ref[...] = jnp.zeros_like(out_ref)

    barrier = pltpu.get_barrier_semaphore()
    for nbr in (left, right):
        pl.semaphore_signal(barrier, inc=1, device_id=(nbr,),
                            device_id_type=pl.DeviceIdType.MESH)
    pl.semaphore_wait(barrier, 2)

    def hop(h):
        wq_rdma = pltpu.make_async_remote_copy(
            src_ref=wq_buf.at[(4 - h) % 4],
            dst_ref=wq_buf.at[(3 - h) % 4],
            send_sem=wq_send.at[h],
            recv_sem=wq_recv.at[h],
            device_id=(right,),
            device_id_type=pl.DeviceIdType.MESH,
        )
        wo_rdma = pltpu.make_async_remote_copy(
            src_ref=wo_buf.at[h],
            dst_ref=wo_buf.at[h + 1],
            send_sem=wo_send.at[h],
            recv_sem=wo_recv.at[h],
            device_id=(left,),
            device_id_type=pl.DeviceIdType.MESH,
        )
        wq_rdma.start()
        wo_rdma.start()
        return wq_rdma, wo_rdma

    def compute(s):
        g = lax.rem(my + s, N_DEV)
        h0 = g * 8
        qg = jnp.dot(x_ref[...], wq_buf[s],
                     preferred_element_type=jnp.float32
                     ).astype(jnp.bfloat16)
        qg5 = qg.reshape(8, 4, 64, 8, 128)
        wo3 = wo_buf[s].reshape(8, 128, 1024)
        for r in range(4):
            ck = pltpu.make_async_copy(
                k_hbm.at[my, :, r, :, pl.ds(h0, 8), :], k_stage, ksem)
            cv = pltpu.make_async_copy(
                v_hbm.at[my, :, r, :, pl.ds(h0, 8), :], v_stage, vsem)
            ck.start()
            cv.start()
            ck.wait()
            cv.wait()
            o = jnp.zeros((512, 1024), jnp.float32)
            for hb in range(2):
                hs = slice(hb * 4, hb * 4 + 4)
                qr = jnp.swapaxes(
                    qg5[:, r, :, hs, :].reshape(512, 4, 128), 0, 1)
                kr = jnp.swapaxes(
                    k_stage[:, :, hs, :].astype(jnp.bfloat16)
                    .reshape(512, 4, 128), 0, 1)
                vr = jnp.swapaxes(
                    v_stage[:, :, hs, :].astype(jnp.bfloat16)
                    .reshape(512, 4, 128), 0, 1)
                sc = lax.dot_general(
                    qr, kr, (((2,), (2,)), ((0,), (0,))),
                    preferred_element_type=jnp.float32) * SCALE
                m = jnp.max(sc, axis=-1, keepdims=True)
                e = jnp.exp(sc - m)
                den = jnp.sum(e, axis=-1, keepdims=True)
                w = (e / den).astype(jnp.bfloat16)
                ctx = lax.dot_general(
                    w, vr, (((2,), (1,)), ((0,), (0,))),
                    preferred_element_type=jnp.float32
                    ).astype(jnp.bfloat16)
                for hh in range(4):
                    o = o + jnp.dot(ctx[hh], wo3[hb * 4 + hh],
                                    preferred_element_type=jnp.float32)
            out_ref[:, r] = out_ref[:, r] + o.reshape(8, 64, 1024)

    h0p = hop(0)
    compute(0)
    h0p[0].wait()
    h0p[1].wait()
    h1p = hop(1)
    h1p[0].wait()
    h1p[1].wait()
    h2p = hop(2)
    compute(2)
    h2p[0].wait()
    h2p[1].wait()
    compute(1)
    compute(3)


def kernel(x, Wq, K_ext, V_ext, Wo):
    xb = x[0].astype(jnp.bfloat16)
    wqb = Wq.astype(jnp.bfloat16)
    wob = Wo.astype(jnp.bfloat16)
    k6 = K_ext.reshape(4, 8, 4, 64, 32, 128)
    v6 = V_ext.reshape(4, 8, 4, 64, 32, 128)

    out = pl.pallas_call(
        _body,
        out_shape=jax.ShapeDtypeStruct((8, 4, 64, 1024), jnp.float32),
        in_specs=[
            pl.BlockSpec(memory_space=pltpu.MemorySpace.VMEM),
            pl.BlockSpec(memory_space=pltpu.MemorySpace.VMEM),
            pl.BlockSpec(memory_space=pltpu.MemorySpace.VMEM),
            pl.BlockSpec(memory_space=pltpu.MemorySpace.HBM),
            pl.BlockSpec(memory_space=pltpu.MemorySpace.HBM),
        ],
        out_specs=pl.BlockSpec(memory_space=pltpu.MemorySpace.VMEM),
        scratch_shapes=[
            pltpu.VMEM((4, 1024, 1024), jnp.bfloat16),
            pltpu.VMEM((4, 1024, 1024), jnp.bfloat16),
            pltpu.VMEM((8, 64, 8, 128), jnp.float32),
            pltpu.VMEM((8, 64, 8, 128), jnp.float32),
            pltpu.SemaphoreType.DMA((3,)),
            pltpu.SemaphoreType.DMA((3,)),
            pltpu.SemaphoreType.DMA((3,)),
            pltpu.SemaphoreType.DMA((3,)),
            pltpu.SemaphoreType.DMA,
            pltpu.SemaphoreType.DMA,
        ],
        compiler_params=pltpu.CompilerParams(
            collective_id=0,
            vmem_limit_bytes=60 * 1024 * 1024,
        ),
    )(xb, wqb, wob, k6, v6)
    return out.reshape(1, 2048, 1024)


# device time: 167738 ns/iter; 1.2671x vs baseline; 1.2671x over previous
import jax
import jax.numpy as jnp
from jax import lax
from jax.experimental import pallas as pl
from jax.experimental.pallas import tpu as pltpu

N_DEV = 4
SCALE = 0.08838834764831843


def _body(x_ref, wq_ref, wo_ref, k_hbm, v_hbm, out_ref,
          wq_buf, wo_buf, k_stage, v_stage, ctx_buf,
          wq_send, wq_recv, wo_send, wo_recv, ksem, vsem):
    my = lax.axis_index("i")
    right = lax.rem(my + 1, N_DEV)
    left = lax.rem(my + 3, N_DEV)

    wq_buf[0] = wq_ref[...]
    wo_buf[0] = wo_ref[...]
    out_ref[...] = jnp.zeros_like(out_ref)

    barrier = pltpu.get_barrier_semaphore()
    for nbr in (left, right):
        pl.semaphore_signal(barrier, inc=1, device_id=(nbr,),
                            device_id_type=pl.DeviceIdType.MESH)
    pl.semaphore_wait(barrier, 2)

    def hop(h):
        wq_rdma = pltpu.make_async_remote_copy(
            src_ref=wq_buf.at[(4 - h) % 4],
            dst_ref=wq_buf.at[(3 - h) % 4],
            send_sem=wq_send.at[h],
            recv_sem=wq_recv.at[h],
            device_id=(right,),
            device_id_type=pl.DeviceIdType.MESH,
        )
        wo_rdma = pltpu.make_async_remote_copy(
            src_ref=wo_buf.at[h],
            dst_ref=wo_buf.at[h + 1],
            send_sem=wo_send.at[h],
            recv_sem=wo_recv.at[h],
            device_id=(left,),
            device_id_type=pl.DeviceIdType.MESH,
        )
        wq_rdma.start()
        wo_rdma.start()
        return wq_rdma, wo_rdma

    order = [(s, r) for s in (0, 2, 1, 3) for r in range(4)]

    def issue(j):
        s, r = order[j]
        h0 = lax.rem(my + s, N_DEV) * 8
        b = j % 2
        ck = pltpu.make_async_copy(
            k_hbm.at[my, :, r, :, pl.ds(h0, 8), :], k_stage.at[b],
            ksem.at[b])
        cv = pltpu.make_async_copy(
            v_hbm.at[my, :, r, :, pl.ds(h0, 8), :], v_stage.at[b],
            vsem.at[b])
        ck.start()
        cv.start()
        return ck, cv

    pend = {0: issue(0)}

    def compute(s):
        j0 = order.index((s, 0))
        qg = jnp.dot(x_ref[...], wq_buf[s],
                     preferred_element_type=jnp.float32
                     ).astype(jnp.bfloat16)
        qg4 = qg.reshape(8, 4, 64, 1024)
        wo_s = wo_buf[s]
        for r in range(4):
            j = j0 + r
            ck, cv = pend.pop(j)
            ck.wait()
            cv.wait()
            if j + 1 < 16:
                pend[j + 1] = issue(j + 1)
            b = j % 2
            qr = qg4[:, r].reshape(512, 1024)
            kr = k_stage[b].astype(jnp.bfloat16)
            vr = v_stage[b].astype(jnp.bfloat16)
            for h in range(8):
                q_h = qr[:, h * 128:(h + 1) * 128]
                k_h = kr[:, :, h, :].reshape(512, 128)
                v_h = vr[:, :, h, :].reshape(512, 128)
                sc = lax.dot_general(
                    q_h, k_h, (((1,), (1,)), ((), ())),
                    preferred_element_type=jnp.float32) * SCALE
                e = jnp.exp(sc)
                rden = 1.0 / jnp.sum(e, axis=-1, keepdims=True)
                w = (e * rden).astype(jnp.bfloat16)
                ctx_buf[:, h * 128:(h + 1) * 128] = lax.dot_general(
                    w, v_h, (((1,), (0,)), ((), ())),
                    preferred_element_type=jnp.float32
                    ).astype(jnp.bfloat16)
            o = jnp.dot(ctx_buf[...], wo_s,
                        preferred_element_type=jnp.float32)
            out_ref[:, r] = out_ref[:, r] + o.reshape(8, 64, 1024)

    h0p = hop(0)
    compute(0)
    h0p[0].wait()
    h0p[1].wait()
    h1p = hop(1)
    h1p[0].wait()
    h1p[1].wait()
    h2p = hop(2)
    compute(2)
    h2p[0].wait()
    h2p[1].wait()
    compute(1)
    compute(3)


def kernel(x, Wq, K_ext, V_ext, Wo):
    xb = x[0].astype(jnp.bfloat16)
    wqb = Wq.astype(jnp.bfloat16)
    wob = Wo.astype(jnp.bfloat16)
    k6 = K_ext.reshape(4, 8, 4, 64, 32, 128)
    v6 = V_ext.reshape(4, 8, 4, 64, 32, 128)

    out = pl.pallas_call(
        _body,
        out_shape=jax.ShapeDtypeStruct((8, 4, 64, 1024), jnp.float32),
        in_specs=[
            pl.BlockSpec(memory_space=pltpu.MemorySpace.VMEM),
            pl.BlockSpec(memory_space=pltpu.MemorySpace.VMEM),
            pl.BlockSpec(memory_space=pltpu.MemorySpace.VMEM),
            pl.BlockSpec(memory_space=pltpu.MemorySpace.HBM),
            pl.BlockSpec(memory_space=pltpu.MemorySpace.HBM),
        ],
        out_specs=pl.BlockSpec(memory_space=pltpu.MemorySpace.VMEM),
        scratch_shapes=[
            pltpu.VMEM((4, 1024, 1024), jnp.bfloat16),
            pltpu.VMEM((4, 1024, 1024), jnp.bfloat16),
            pltpu.VMEM((2, 8, 64, 8, 128), jnp.float32),
            pltpu.VMEM((2, 8, 64, 8, 128), jnp.float32),
            pltpu.VMEM((512, 1024), jnp.bfloat16),
            pltpu.SemaphoreType.DMA((3,)),
            pltpu.SemaphoreType.DMA((3,)),
            pltpu.SemaphoreType.DMA((3,)),
            pltpu.SemaphoreType.DMA((3,)),
            pltpu.SemaphoreType.DMA((2,)),
            pltpu.SemaphoreType.DMA((2,)),
        ],
        compiler_params=pltpu.CompilerParams(
            collective_id=0,
            vmem_limit_bytes=60 * 1024 * 1024,
        ),
    )(xb, wqb, wob, k6, v6)
    return out.reshape(1, 2048, 1024)


# device time: 133967 ns/iter; 1.5865x vs baseline; 1.2521x over previous
import jax
import jax.numpy as jnp
from jax import lax
from jax.experimental import pallas as pl
from jax.experimental.pallas import tpu as pltpu

N_DEV = 4
SCALE = 0.08838834764831843


def _body(x_ref, wq_ref, wo_ref, k_hbm, v_hbm, out_ref,
          wq_buf, wo_buf, k_stage, v_stage, ctx_buf, ctx_dfr,
          wq_send, wq_recv, wo_send, wo_recv, ksem, vsem):
    my = lax.axis_index("i")
    right = lax.rem(my + 1, N_DEV)
    left = lax.rem(my + 3, N_DEV)

    wq_buf[0] = wq_ref[...]
    wo_buf[0] = wo_ref[...]
    out_ref[...] = jnp.zeros_like(out_ref)

    barrier = pltpu.get_barrier_semaphore()
    for nbr in (left, right):
        pl.semaphore_signal(barrier, inc=1, device_id=(nbr,),
                            device_id_type=pl.DeviceIdType.MESH)
    pl.semaphore_wait(barrier, 2)

    def hop(h):
        wq_rdma = pltpu.make_async_remote_copy(
            src_ref=wq_buf.at[(4 - h) % 4],
            dst_ref=wq_buf.at[(3 - h) % 4],
            send_sem=wq_send.at[h],
            recv_sem=wq_recv.at[h],
            device_id=(right,),
            device_id_type=pl.DeviceIdType.MESH,
        )
        wo_rdma = pltpu.make_async_remote_copy(
            src_ref=wo_buf.at[h],
            dst_ref=wo_buf.at[h + 1],
            send_sem=wo_send.at[h],
            recv_sem=wo_recv.at[h],
            device_id=(left,),
            device_id_type=pl.DeviceIdType.MESH,
        )
        wq_rdma.start()
        wo_rdma.start()
        return wq_rdma, wo_rdma

    order = [(s, r) for s in (0, 3, 2, 1) for r in range(4)]

    def issue(j):
        s, r = order[j]
        h0 = lax.rem(my + s, N_DEV) * 8
        b = j % 2
        ck = pltpu.make_async_copy(
            k_hbm.at[my, :, r, :, pl.ds(h0, 8), :], k_stage.at[b],
            ksem.at[b])
        cv = pltpu.make_async_copy(
            v_hbm.at[my, :, r, :, pl.ds(h0, 8), :], v_stage.at[b],
            vsem.at[b])
        ck.start()
        cv.start()
        return ck, cv

    pend = {0: issue(0)}

    def wo_apply(s, r, ctx2d):
        o = jnp.dot(ctx2d, wo_buf[s],
                    preferred_element_type=jnp.float32)
        out_ref[:, r] = out_ref[:, r] + o.reshape(8, 64, 1024)

    def attn_slot(s, defer):
        j0 = order.index((s, 0))
        qg = (jnp.dot(x_ref[...], wq_buf[s],
                      preferred_element_type=jnp.float32)
              * SCALE).astype(jnp.bfloat16)
        qg4 = qg.reshape(8, 4, 64, 1024)
        for r in range(4):
            j = j0 + r
            ck, cv = pend.pop(j)
            ck.wait()
            cv.wait()
            if j + 1 < 16:
                pend[j + 1] = issue(j + 1)
            b = j % 2
            qr = qg4[:, r].reshape(512, 1024)
            kr = k_stage[b].astype(jnp.bfloat16)
            vr = v_stage[b].astype(jnp.bfloat16)
            for h in range(8):
                q_h = qr[:, h * 128:(h + 1) * 128]
                k_h = kr[:, :, h, :].reshape(512, 128)
                v_h = vr[:, :, h, :].reshape(512, 128)
                sc = lax.dot_general(
                    q_h, k_h, (((1,), (1,)), ((), ())),
                    preferred_element_type=jnp.float32)
                e = jnp.exp(sc)
                rden = 1.0 / jnp.sum(e, axis=-1, keepdims=True)
                ctx = lax.dot_general(
                    e.astype(jnp.bfloat16), v_h, (((1,), (0,)), ((), ())),
                    preferred_element_type=jnp.float32)
                ctx = (ctx * rden).astype(jnp.bfloat16)
                if defer:
                    ctx_dfr[r, :, h * 128:(h + 1) * 128] = ctx
                else:
                    ctx_buf[:, h * 128:(h + 1) * 128] = ctx
            if not defer:
                wo_apply(s, r, ctx_buf[...])

    h0p = hop(0)
    attn_slot(0, defer=False)
    h0p[0].wait()
    h0p[1].wait()
    h1p = hop(1)
    attn_slot(3, defer=True)
    h1p[0].wait()
    h1p[1].wait()
    h2p = hop(2)
    attn_slot(2, defer=False)
    h2p[0].wait()
    h2p[1].wait()
    for r in range(4):
        wo_apply(3, r, ctx_dfr[r])
    attn_slot(1, defer=False)


def kernel(x, Wq, K_ext, V_ext, Wo):
    xb = x[0].astype(jnp.bfloat16)
    wqb = Wq.astype(jnp.bfloat16)
    wob = Wo.astype(jnp.bfloat16)
    k6 = K_ext.reshape(4, 8, 4, 64, 32, 128)
    v6 = V_ext.reshape(4, 8, 4, 64, 32, 128)

    out = pl.pallas_call(
        _body,
        out_shape=jax.ShapeDtypeStruct((8, 4, 64, 1024), jnp.float32),
        in_specs=[
            pl.BlockSpec(memory_space=pltpu.MemorySpace.VMEM),
            pl.BlockSpec(memory_space=pltpu.MemorySpace.VMEM),
            pl.BlockSpec(memory_space=pltpu.MemorySpace.VMEM),
            pl.BlockSpec(memory_space=pltpu.MemorySpace.HBM),
            pl.BlockSpec(memory_space=pltpu.MemorySpace.HBM),
        ],
        out_specs=pl.BlockSpec(memory_space=pltpu.MemorySpace.VMEM),
        scratch_shapes=[
            pltpu.VMEM((4, 1024, 1024), jnp.bfloat16),
            pltpu.VMEM((4, 1024, 1024), jnp.bfloat16),
            pltpu.VMEM((2, 8, 64, 8, 128), jnp.float32),
            pltpu.VMEM((2, 8, 64, 8, 128), jnp.float32),
            pltpu.VMEM((512, 1024), jnp.bfloat16),
            pltpu.VMEM((4, 512, 1024), jnp.bfloat16),
            pltpu.SemaphoreType.DMA((3,)),
            pltpu.SemaphoreType.DMA((3,)),
            pltpu.SemaphoreType.DMA((3,)),
            pltpu.SemaphoreType.DMA((3,)),
            pltpu.SemaphoreType.DMA((2,)),
            pltpu.SemaphoreType.DMA((2,)),
        ],
        compiler_params=pltpu.CompilerParams(
            collective_id=0,
            vmem_limit_bytes=60 * 1024 * 1024,
        ),
    )(xb, wqb, wob, k6, v6)
    return out.reshape(1, 2048, 1024)
